# Initial kernel scaffold; baseline (speedup 1.0000x reference)
import functools

import jax
import jax.numpy as jnp
from jax import lax
from jax.experimental import pallas as pl
from jax.experimental.pallas import tpu as pltpu, tpu_sc as plsc

B = 4096
L = 200
D = 32
OUT_D = 3 * D + 1
N = B * L

NUM_WORKERS = 32
PER_W = N // NUM_WORKERS
CHUNK = 512
SUB = 128
NSUB = CHUNK // SUB
NCHUNKS = PER_W // CHUNK


def _body(goods_t, shop_t, cate_t, prices, gids, sids, cids, out,
          idx_g, idx_s, idx_c, gbuf, sbuf, cbuf, pbuf, gsem, wsem):
    wid = lax.axis_index("s") * 2 + lax.axis_index("c")
    w_base = wid * PER_W

    @pl.loop(0, NCHUNKS)
    def _chunk(g):
        base = w_base + g * CHUNK
        row = base // SUB

        pltpu.sync_copy(gids.at[pl.ds(row, NSUB)], idx_g)
        pltpu.sync_copy(sids.at[pl.ds(row, NSUB)], idx_s)
        pltpu.sync_copy(cids.at[pl.ds(row, NSUB)], idx_c)
        pltpu.sync_copy(prices.at[pl.ds(base, CHUNK)], pbuf)

        copies = []
        for j in range(NSUB):
            dst = pl.ds(j * SUB, SUB)
            copies.append(pltpu.async_copy(goods_t.at[idx_g.at[j]],
                                           gbuf.at[dst], gsem))
            copies.append(pltpu.async_copy(shop_t.at[idx_s.at[j]],
                                           sbuf.at[dst], gsem))
            copies.append(pltpu.async_copy(cate_t.at[idx_c.at[j]],
                                           cbuf.at[dst], gsem))
        for c in copies:
            c.wait()

        rows = pl.ds(base, CHUNK)
        w = [pltpu.async_copy(gbuf, out.at[rows, pl.ds(0, D)], wsem),
             pltpu.async_copy(sbuf, out.at[rows, pl.ds(D, D)], wsem),
             pltpu.async_copy(cbuf, out.at[rows, pl.ds(2 * D, D)], wsem),
             pltpu.async_copy(pbuf, out.at[rows, pl.ds(3 * D, 1)], wsem)]
        for c in w:
            c.wait()


@jax.jit
def _sc_lookup(goods_t, shop_t, cate_t, prices2d, gids2d, sids2d, cids2d):
    mesh = plsc.VectorSubcoreMesh(core_axis_name="c", subcore_axis_name="s")
    return pl.kernel(
        _body,
        out_type=jax.ShapeDtypeStruct((N, OUT_D), jnp.float32),
        mesh=mesh,
        scratch_types=[
            pltpu.VMEM((NSUB, SUB), jnp.int32),
            pltpu.VMEM((NSUB, SUB), jnp.int32),
            pltpu.VMEM((NSUB, SUB), jnp.int32),
            pltpu.VMEM((CHUNK, D), jnp.float32),
            pltpu.VMEM((CHUNK, D), jnp.float32),
            pltpu.VMEM((CHUNK, D), jnp.float32),
            pltpu.VMEM((CHUNK, 1), jnp.float32),
            pltpu.SemaphoreType.DMA,
            pltpu.SemaphoreType.DMA,
        ],
    )(goods_t, shop_t, cate_t, prices2d, gids2d, sids2d, cids2d)


def kernel(goods_table, shop_table, cate_table, goods_prices,
           goods_ids, shop_ids, cate_ids):
    gids2d = goods_ids.reshape(N // SUB, SUB).astype(jnp.int32)
    sids2d = shop_ids.reshape(N // SUB, SUB).astype(jnp.int32)
    cids2d = cate_ids.reshape(N // SUB, SUB).astype(jnp.int32)
    prices2d = goods_prices.reshape(N, 1)
    out = _sc_lookup(goods_table, shop_table, cate_table,
                     prices2d, gids2d, sids2d, cids2d)
    return out.reshape(B, L, OUT_D)


# speedup vs baseline: 3.1095x; 3.1095x over previous
import functools

import jax
import jax.numpy as jnp
from jax import lax
from jax.experimental import pallas as pl
from jax.experimental.pallas import tpu as pltpu, tpu_sc as plsc

B = 4096
L = 200
D = 32
OUT_D = 3 * D + 1
N = B * L

NUM_WORKERS = 32
PER_W = N // NUM_WORKERS
CHUNK = 512
SUB = 128
NSUB = CHUNK // SUB
NCHUNKS = PER_W // CHUNK


def _body(goods_t, shop_t, cate_t, prices, gids, sids, cids, out,
          idx_g, idx_s, idx_c, gbuf, sbuf, cbuf, pbuf, gsem, wsem):
    wid = lax.axis_index("s") * 2 + lax.axis_index("c")
    w_base = wid * PER_W

    @pl.loop(0, NCHUNKS)
    def _chunk(g):
        base = w_base + g * CHUNK
        row = base // SUB

        pltpu.sync_copy(gids.at[pl.ds(row, NSUB)], idx_g)
        pltpu.sync_copy(sids.at[pl.ds(row, NSUB)], idx_s)
        pltpu.sync_copy(cids.at[pl.ds(row, NSUB)], idx_c)
        pltpu.sync_copy(prices.at[pl.ds(base, CHUNK)], pbuf)

        copies = []
        for j in range(NSUB):
            dst = pl.ds(j * SUB, SUB)
            copies.append(pltpu.async_copy(goods_t.at[idx_g.at[j]],
                                           gbuf.at[dst], gsem))
            copies.append(pltpu.async_copy(shop_t.at[idx_s.at[j]],
                                           sbuf.at[dst], gsem))
            copies.append(pltpu.async_copy(cate_t.at[idx_c.at[j]],
                                           cbuf.at[dst], gsem))
        for c in copies:
            c.wait()

        rows = pl.ds(base, CHUNK)
        w = [pltpu.async_copy(gbuf, out.at[rows, pl.ds(0, D)], wsem),
             pltpu.async_copy(sbuf, out.at[rows, pl.ds(D, D)], wsem),
             pltpu.async_copy(cbuf, out.at[rows, pl.ds(2 * D, D)], wsem),
             pltpu.async_copy(pbuf, out.at[rows, pl.ds(3 * D, 1)], wsem)]
        for c in w:
            c.wait()


@jax.jit
def _sc_lookup(goods_t, shop_t, cate_t, prices2d, gids2d, sids2d, cids2d):
    mesh = plsc.VectorSubcoreMesh(core_axis_name="c", subcore_axis_name="s")
    return pl.kernel(
        _body,
        out_type=jax.ShapeDtypeStruct((N, OUT_D), jnp.float32),
        mesh=mesh,
        compiler_params=pltpu.CompilerParams(use_tc_tiling_on_sc=False),
        scratch_types=[
            pltpu.VMEM((NSUB, SUB), jnp.int32),
            pltpu.VMEM((NSUB, SUB), jnp.int32),
            pltpu.VMEM((NSUB, SUB), jnp.int32),
            pltpu.VMEM((CHUNK, D), jnp.float32),
            pltpu.VMEM((CHUNK, D), jnp.float32),
            pltpu.VMEM((CHUNK, D), jnp.float32),
            pltpu.VMEM((CHUNK, 1), jnp.float32),
            pltpu.SemaphoreType.DMA,
            pltpu.SemaphoreType.DMA,
        ],
    )(goods_t, shop_t, cate_t, prices2d, gids2d, sids2d, cids2d)


def kernel(goods_table, shop_table, cate_table, goods_prices,
           goods_ids, shop_ids, cate_ids):
    gids2d = goods_ids.reshape(N // SUB, SUB).astype(jnp.int32)
    sids2d = shop_ids.reshape(N // SUB, SUB).astype(jnp.int32)
    cids2d = cate_ids.reshape(N // SUB, SUB).astype(jnp.int32)
    prices2d = goods_prices.reshape(N, 1)
    out = _sc_lookup(goods_table, shop_table, cate_table,
                     prices2d, gids2d, sids2d, cids2d)
    return out.reshape(B, L, OUT_D)


# speedup vs baseline: 3.2531x; 1.0462x over previous
import jax
import jax.numpy as jnp
from jax import lax
from jax.experimental import pallas as pl
from jax.experimental.pallas import tpu as pltpu, tpu_sc as plsc

B = 4096
L = 200
D = 32
OUT_D = 3 * D + 1
N = B * L

NUM_WORKERS = 32
PER_W = N // NUM_WORKERS
CHUNK = 512
SUB = 128
NSUB = CHUNK // SUB
PAIR = 2 * CHUNK
RPP = PAIR // SUB
NPAIRS = PER_W // PAIR


def _body(goods_t, shop_t, cate_t, prices, gids, sids, cids, out,
          idxg, idxs, idxc, pbuf, g0, s0, c0, g1, s1, c1,
          isem, gsem, wsem):
    wid = lax.axis_index("s") * 2 + lax.axis_index("c")
    w_base = wid * PER_W

    def id_copies(p, sl):
        pair_base = w_base + p * PAIR
        rows = pl.ds(pair_base // SUB, RPP)
        return [pltpu.make_async_copy(gids.at[rows], idxg.at[sl], isem),
                pltpu.make_async_copy(sids.at[rows], idxs.at[sl], isem),
                pltpu.make_async_copy(cids.at[rows], idxc.at[sl], isem),
                pltpu.make_async_copy(prices.at[pl.ds(pair_base, PAIR)],
                                      pbuf.at[sl], isem)]

    def gather_copies(ci, sl, gb, sb, cb):
        cps = []
        for j in range(NSUB):
            r = ci * NSUB + j
            rows = pl.ds(j * SUB, SUB)
            cps += [pltpu.make_async_copy(goods_t.at[idxg.at[sl, r]],
                                          gb.at[rows], gsem),
                    pltpu.make_async_copy(shop_t.at[idxs.at[sl, r]],
                                          sb.at[rows], gsem),
                    pltpu.make_async_copy(cate_t.at[idxc.at[sl, r]],
                                          cb.at[rows], gsem)]
        return cps

    def write_copies(p, ci, sl, gb, sb, cb):
        base = w_base + p * PAIR + ci * CHUNK
        rows = pl.ds(base, CHUNK)
        return [pltpu.make_async_copy(gb, out.at[rows, pl.ds(0, D)], wsem),
                pltpu.make_async_copy(sb, out.at[rows, pl.ds(D, D)], wsem),
                pltpu.make_async_copy(cb, out.at[rows, pl.ds(2 * D, D)],
                                      wsem),
                pltpu.make_async_copy(pbuf.at[sl, pl.ds(ci * CHUNK, CHUNK)],
                                      out.at[rows, pl.ds(3 * D, 1)], wsem)]

    for c in id_copies(0, 0):
        c.start()

    @pl.loop(0, NPAIRS)
    def _pair(p):
        sl = lax.rem(p, 2)
        for c in id_copies(p, sl):
            c.wait()

        @pl.when(p > 0)
        def _drain0():
            for c in write_copies(p - 1, 0, 1 - sl, g0, s0, c0):
                c.wait()
        for c in gather_copies(0, sl, g0, s0, c0):
            c.start()

        @pl.when(p > 0)
        def _drain1():
            for c in write_copies(p - 1, 1, 1 - sl, g1, s1, c1):
                c.wait()
        for c in gather_copies(1, sl, g1, s1, c1):
            c.start()

        @pl.when(p < NPAIRS - 1)
        def _prefetch():
            for c in id_copies(p + 1, 1 - sl):
                c.start()

        for c in gather_copies(0, sl, g0, s0, c0):
            c.wait()
        for c in write_copies(p, 0, sl, g0, s0, c0):
            c.start()
        for c in gather_copies(1, sl, g1, s1, c1):
            c.wait()
        for c in write_copies(p, 1, sl, g1, s1, c1):
            c.start()

    last = NPAIRS - 1
    lsl = lax.rem(last, 2)
    for c in write_copies(last, 0, lsl, g0, s0, c0):
        c.wait()
    for c in write_copies(last, 1, lsl, g1, s1, c1):
        c.wait()


@jax.jit
def _sc_lookup(goods_t, shop_t, cate_t, prices2d, gids2d, sids2d, cids2d):
    mesh = plsc.VectorSubcoreMesh(core_axis_name="c", subcore_axis_name="s")
    return pl.kernel(
        _body,
        out_type=jax.ShapeDtypeStruct((N, OUT_D), jnp.float32),
        mesh=mesh,
        compiler_params=pltpu.CompilerParams(use_tc_tiling_on_sc=False),
        scratch_types=[
            pltpu.VMEM((2, RPP, SUB), jnp.int32),
            pltpu.VMEM((2, RPP, SUB), jnp.int32),
            pltpu.VMEM((2, RPP, SUB), jnp.int32),
            pltpu.VMEM((2, PAIR, 1), jnp.float32),
            pltpu.VMEM((CHUNK, D), jnp.float32),
            pltpu.VMEM((CHUNK, D), jnp.float32),
            pltpu.VMEM((CHUNK, D), jnp.float32),
            pltpu.VMEM((CHUNK, D), jnp.float32),
            pltpu.VMEM((CHUNK, D), jnp.float32),
            pltpu.VMEM((CHUNK, D), jnp.float32),
            pltpu.SemaphoreType.DMA,
            pltpu.SemaphoreType.DMA,
            pltpu.SemaphoreType.DMA,
        ],
    )(goods_t, shop_t, cate_t, prices2d, gids2d, sids2d, cids2d)


def kernel(goods_table, shop_table, cate_table, goods_prices,
           goods_ids, shop_ids, cate_ids):
    gids2d = goods_ids.reshape(N // SUB, SUB).astype(jnp.int32)
    sids2d = shop_ids.reshape(N // SUB, SUB).astype(jnp.int32)
    cids2d = cate_ids.reshape(N // SUB, SUB).astype(jnp.int32)
    prices2d = goods_prices.reshape(N, 1)
    out = _sc_lookup(goods_table, shop_table, cate_table,
                     prices2d, gids2d, sids2d, cids2d)
    return out.reshape(B, L, OUT_D)


# speedup vs baseline: 4.5341x; 1.3938x over previous
import jax
import jax.numpy as jnp
from jax import lax
from jax.experimental import pallas as pl
from jax.experimental.pallas import tpu as pltpu, tpu_sc as plsc

B = 4096
L = 200
D = 32
OUT_D = 3 * D + 1
N = B * L

NUM_WORKERS = 32
PER_W = N // NUM_WORKERS
CHUNK = 512
SUB = 128
NSUB = CHUNK // SUB
PAIR = 2 * CHUNK
RPP = PAIR // SUB
NPAIRS = PER_W // PAIR


def _body(goods_t, shop_t, cate_t, prices, gids, sids, cids, out,
          idxg, idxs, idxc, pbuf, pb20, pb21, g0, s0, c0, g1, s1, c1,
          isem, gsem, wsem):
    wid = lax.axis_index("s") * 2 + lax.axis_index("c")
    w_base = wid * PER_W
    lane = lax.iota(jnp.int32, 16)
    zero16 = jnp.zeros((16,), jnp.int32)

    def id_copies(p, sl):
        pair_base = w_base + p * PAIR
        rows = pl.ds(pair_base // SUB, RPP)
        return [pltpu.make_async_copy(gids.at[rows], idxg.at[sl], isem),
                pltpu.make_async_copy(sids.at[rows], idxs.at[sl], isem),
                pltpu.make_async_copy(cids.at[rows], idxc.at[sl], isem),
                pltpu.make_async_copy(prices.at[pl.ds(pair_base, PAIR)],
                                      pbuf.at[sl], isem)]

    def fill_price(ci, sl, pb2):
        for i in range(CHUNK // 16):
            vals = pbuf[sl, pl.ds(ci * CHUNK + i * 16, 16)]
            plsc.store_scatter(pb2, [lane + i * 16, zero16], vals)

    def gather_copies(ci, sl, gb, sb, cb):
        cps = []
        for j in range(NSUB):
            r = ci * NSUB + j
            rows = pl.ds(j * SUB, SUB)
            cps += [pltpu.make_async_copy(goods_t.at[idxg.at[sl, r]],
                                          gb.at[rows], gsem),
                    pltpu.make_async_copy(shop_t.at[idxs.at[sl, r]],
                                          sb.at[rows], gsem),
                    pltpu.make_async_copy(cate_t.at[idxc.at[sl, r]],
                                          cb.at[rows], gsem)]
        return cps

    def write_copies(p, ci, pb2, gb, sb, cb):
        base = w_base + p * PAIR + ci * CHUNK
        rows = pl.ds(base, CHUNK)
        return [pltpu.make_async_copy(gb, out.at[rows, pl.ds(0, D)], wsem),
                pltpu.make_async_copy(sb, out.at[rows, pl.ds(D, D)], wsem),
                pltpu.make_async_copy(cb, out.at[rows, pl.ds(2 * D, D)],
                                      wsem),
                pltpu.make_async_copy(pb2,
                                      out.at[rows, pl.ds(3 * D, 1)], wsem)]

    for c in id_copies(0, 0):
        c.start()

    @pl.loop(0, NPAIRS)
    def _pair(p):
        sl = lax.rem(p, 2)
        for c in id_copies(p, sl):
            c.wait()

        @pl.when(p > 0)
        def _drain0():
            for c in write_copies(p - 1, 0, pb20, g0, s0, c0):
                c.wait()
        for c in gather_copies(0, sl, g0, s0, c0):
            c.start()
        fill_price(0, sl, pb20)

        @pl.when(p > 0)
        def _drain1():
            for c in write_copies(p - 1, 1, pb21, g1, s1, c1):
                c.wait()
        for c in gather_copies(1, sl, g1, s1, c1):
            c.start()
        fill_price(1, sl, pb21)

        @pl.when(p < NPAIRS - 1)
        def _prefetch():
            for c in id_copies(p + 1, 1 - sl):
                c.start()

        for c in gather_copies(0, sl, g0, s0, c0):
            c.wait()
        for c in write_copies(p, 0, pb20, g0, s0, c0):
            c.start()
        for c in gather_copies(1, sl, g1, s1, c1):
            c.wait()
        for c in write_copies(p, 1, pb21, g1, s1, c1):
            c.start()

    last = NPAIRS - 1
    for c in write_copies(last, 0, pb20, g0, s0, c0):
        c.wait()
    for c in write_copies(last, 1, pb21, g1, s1, c1):
        c.wait()


@jax.jit
def _sc_lookup(goods_t, shop_t, cate_t, prices1d, gids2d, sids2d, cids2d):
    mesh = plsc.VectorSubcoreMesh(core_axis_name="c", subcore_axis_name="s")
    return pl.kernel(
        _body,
        out_type=jax.ShapeDtypeStruct((N, OUT_D), jnp.float32),
        mesh=mesh,
        compiler_params=pltpu.CompilerParams(use_tc_tiling_on_sc=False,
                                            needs_layout_passes=False),
        scratch_types=[
            pltpu.VMEM((2, RPP, SUB), jnp.int32),
            pltpu.VMEM((2, RPP, SUB), jnp.int32),
            pltpu.VMEM((2, RPP, SUB), jnp.int32),
            pltpu.VMEM((2, PAIR), jnp.float32),
            pltpu.VMEM((CHUNK, 1), jnp.float32),
            pltpu.VMEM((CHUNK, 1), jnp.float32),
            pltpu.VMEM((CHUNK, D), jnp.float32),
            pltpu.VMEM((CHUNK, D), jnp.float32),
            pltpu.VMEM((CHUNK, D), jnp.float32),
            pltpu.VMEM((CHUNK, D), jnp.float32),
            pltpu.VMEM((CHUNK, D), jnp.float32),
            pltpu.VMEM((CHUNK, D), jnp.float32),
            pltpu.SemaphoreType.DMA,
            pltpu.SemaphoreType.DMA,
            pltpu.SemaphoreType.DMA,
        ],
    )(goods_t, shop_t, cate_t, prices1d, gids2d, sids2d, cids2d)


def kernel(goods_table, shop_table, cate_table, goods_prices,
           goods_ids, shop_ids, cate_ids):
    gids2d = goods_ids.reshape(N // SUB, SUB).astype(jnp.int32)
    sids2d = shop_ids.reshape(N // SUB, SUB).astype(jnp.int32)
    cids2d = cate_ids.reshape(N // SUB, SUB).astype(jnp.int32)
    prices1d = goods_prices.reshape(N)
    out = _sc_lookup(goods_table, shop_table, cate_table,
                     prices1d, gids2d, sids2d, cids2d)
    return out.reshape(B, L, OUT_D)
